# baseline (speedup 1.0000x reference)
import functools

import jax
import jax.numpy as jnp
from jax import lax
from jax.experimental import pallas as pl
from jax.experimental.pallas import tpu as pltpu
from jax.experimental.pallas import tpu_sc as plsc

VOCAB = 100000
D_MODEL = 768
BATCH = 4
SEQ = 8192

_NSLOTS = 4
_POS_PER_CHUNK = 4


def _make_embed(vocab, d, batch, seq, num_cores=2, num_subcores=16):
  nw = num_cores * num_subcores
  assert seq % nw == 0
  pos_per_w = seq // nw
  pc = _POS_PER_CHUNK
  assert pos_per_w % (_NSLOTS * pc) == 0
  n_chunks = pos_per_w // pc
  n_rounds = n_chunks // _NSLOTS
  rows_per_chunk = batch * pc
  idx_per_w = batch * pos_per_w
  lanes_per_row = d // 16
  assert d % 16 == 0
  n_rows = batch * seq

  mesh = plsc.VectorSubcoreMesh(core_axis_name="c", subcore_axis_name="s",
                                num_cores=num_cores,
                                num_subcores=num_subcores)

  @functools.partial(
      pl.kernel,
      out_type=jax.ShapeDtypeStruct((n_rows, d), jnp.float32),
      mesh=mesh,
      scratch_types=[
          pltpu.VMEM((idx_per_w,), jnp.int32),
          [pltpu.VMEM((rows_per_chunk, d), jnp.float32)] * _NSLOTS,
          [pltpu.VMEM((pc, d), jnp.float32)] * _NSLOTS,
          [pltpu.SemaphoreType.DMA] * _NSLOTS,
          [pltpu.SemaphoreType.DMA] * _NSLOTS,
          [pltpu.SemaphoreType.DMA] * _NSLOTS,
      ],
  )
  def embed(xp_hbm, tok_hbm, pos_hbm, out_hbm, idx_v,
            toks, poss, tsems, psems, osems):
    wid = lax.axis_index("s") * num_cores + lax.axis_index("c")
    pos_base = wid * pos_per_w
    pltpu.sync_copy(xp_hbm.at[pl.ds(wid * idx_per_w, idx_per_w)], idx_v)

    def start_in(g, slot):
      pltpu.async_copy(
          tok_hbm.at[idx_v.at[pl.ds(g * rows_per_chunk, rows_per_chunk)]],
          toks[slot], tsems[slot])
      pltpu.async_copy(
          pos_hbm.at[pl.ds(pos_base + g * pc, pc)], poss[slot], psems[slot])

    def wait_in(g, slot):
      pltpu.make_async_copy(
          tok_hbm.at[idx_v.at[pl.ds(g * rows_per_chunk, rows_per_chunk)]],
          toks[slot], tsems[slot]).wait()
      pltpu.make_async_copy(
          pos_hbm.at[pl.ds(pos_base + g * pc, pc)], poss[slot],
          psems[slot]).wait()

    def compute(slot):
      tok_v = toks[slot]
      pos_v = poss[slot]

      def add_pos(r, carry):
        for b in range(batch):
          for j in range(lanes_per_row):
            o = j * 16
            plsc.addupdate(tok_v.at[b * pc + r, pl.ds(o, 16)],
                           pos_v[r, pl.ds(o, 16)])
        return carry

      lax.fori_loop(0, pc, add_pos, 0)

    def out_slices(g, slot):
      for b in range(batch):
        yield (toks[slot].at[pl.ds(b * pc, pc)],
               out_hbm.at[pl.ds(b * seq + pos_base + g * pc, pc)])

    def start_out(g, slot):
      for src, dst in out_slices(g, slot):
        pltpu.async_copy(src, dst, osems[slot])

    def wait_out(g, slot):
      for src, dst in out_slices(g, slot):
        pltpu.make_async_copy(src, dst, osems[slot]).wait()

    for b in range(_NSLOTS - 1):
      start_in(b, b)

    def round_(q, carry):
      for b in range(_NSLOTS):
        g = q * _NSLOTS + b
        wait_in(g, b)
        compute(b)
        start_out(g, b)
        s = (b + _NSLOTS - 1) % _NSLOTS
        gc = g + _NSLOTS - 1

        @pl.when(gc < n_chunks)
        def _():
          @pl.when(gc >= _NSLOTS)
          def _():
            wait_out(gc - _NSLOTS, s)

          start_in(gc, s)

      return carry

    lax.fori_loop(0, n_rounds, round_, 0)

    for b in range(_NSLOTS):
      wait_out(n_chunks - _NSLOTS + b, b)

  return embed


_embed_full = _make_embed(VOCAB, D_MODEL, BATCH, SEQ)


@jax.jit
def kernel(x, token_table, pos_table):
  nw = 32
  pos_per_w = SEQ // nw
  pc = _POS_PER_CHUNK
  xp = (x.astype(jnp.int32)
        .reshape(BATCH, nw, pos_per_w // pc, pc)
        .transpose(1, 2, 0, 3)
        .reshape(-1))
  out = _embed_full(xp, token_table, pos_table)
  return out.reshape(BATCH, SEQ, D_MODEL)
